# baseline (speedup 1.0000x reference)
import functools

import jax
import jax.numpy as jnp
import numpy as np
from jax import lax
from jax.experimental import pallas as pl
from jax.experimental.pallas import tpu as pltpu
from jax.experimental.pallas import tpu_sc as plsc

_INV2_23 = np.float32(2.0 ** -23)


def _tf_bits(k1, k2, p):
    ks2 = k1 ^ k2 ^ jnp.uint32(0x1BD11BDA)
    x0 = k1
    x1 = p + k2
    ks_a = (k2, ks2, k1, k2, ks2)
    ks_b = (ks2, k1, k2, ks2, k1)
    rots = ((13, 15, 26, 6), (17, 29, 16, 24))
    for i in range(5):
        for r in rots[i & 1]:
            x0 = x0 + x1
            x1 = (x1 << r) | (x1 >> (32 - r))
            x1 = x0 ^ x1
        x0 = x0 + ks_a[i]
        x1 = x1 + ks_b[i] + jnp.uint32(i + 1)
    return x0 ^ x1


def _bf(x):
    return x.astype(jnp.bfloat16)


def _dot(a, b):
    return lax.dot_general(_bf(a), _bf(b), (((1,), (0,)), ((), ())),
                           preferred_element_type=jnp.float32)


def _prep_body(m, d, ck, sp_ref, noise_ref, lw_ref, obs_ref, ctrl_ref,
               a_ref, b_ref, c_ref, spred_ref, cw_ref, est_ref, lwp_ref):
    cb = _dot(ctrl_ref[0], b_ref[...])
    amat = a_ref[...]
    cmat = c_ref[...]
    obs = obs_ref[0]

    def pass1(t, _):
        sl = pl.ds(t * ck, ck)
        sp = sp_ref[0, sl, :]
        spred = sp + jnp.tanh(_dot(sp, amat) + cb) + noise_ref[0, sl, :]
        spred_ref[0, sl, 0:d] = spred
        po = _dot(spred, cmat)
        df = obs - po
        ll = -0.5 * jnp.sum(df * df, axis=1, keepdims=True)
        lwp_ref[:, sl] = lw_ref[0, :, sl] + jnp.reshape(ll, (1, ck))
        return 0

    lax.fori_loop(0, m // ck, pass1, 0)

    lwp = lwp_ref[...]
    mx = jnp.max(lwp)
    lse = mx + jnp.log(jnp.sum(jnp.exp(lwp - mx)))
    cw_ref[0] = jnp.exp(mx - lwp)

    def pass2(t, acc):
        sl = pl.ds(t * ck, ck)
        w = jnp.exp(lwp_ref[:, sl] - lse)
        return acc + lax.dot_general(
            w, spred_ref[0, sl, 0:d], (((1,), (0,)), ((), ())),
            preferred_element_type=jnp.float32,
            precision=lax.Precision.HIGHEST)

    est_ref[0] = lax.fori_loop(0, m // ck, pass2, jnp.zeros((1, d), jnp.float32))


def _sampler_body(m, bi, isz, js, keys_ref, c_ref, idx_ref):
    r = pl.program_id(0)
    ib = pl.program_id(1)
    k1 = keys_ref[r, 0].astype(jnp.uint32)
    k2 = keys_ref[r, 1].astype(jnp.uint32)
    i0 = ib * bi
    ii = lax.broadcasted_iota(jnp.int32, (isz, js), 0)
    jj = lax.broadcasted_iota(jnp.int32, (isz, js), 1)

    def isub(s, _):
        pbase = ((i0 + s * isz + ii) * m + jj).astype(jnp.uint32)

        def chunk(q, carry):
            rmin, ridx = carry
            p = pbase + (q * js).astype(jnp.uint32)
            bits = _tf_bits(k1, k2, p)
            mant = (bits >> 9).astype(jnp.int32)
            u = mant.astype(jnp.float32) * _INV2_23
            t = jnp.log(u)
            cq = c_ref[0, :, pl.ds(q * js, js)]
            score = t * cq
            cm = jnp.max(score, axis=1, keepdims=True)
            jm = jnp.min(jnp.where(score == cm, jj + q * js, m),
                         axis=1, keepdims=True)
            upd = cm > rmin
            return jnp.where(upd, cm, rmin), jnp.where(upd, jm, ridx)

        rmin0 = jnp.full((isz, 1), -jnp.inf, jnp.float32)
        ridx0 = jnp.zeros((isz, 1), jnp.int32)
        _, ridx = lax.fori_loop(0, m // js, chunk, (rmin0, ridx0))
        idx_ref[0, pl.ds(s * isz, isz), :] = ridx + r * m
        return 0

    lax.fori_loop(0, bi // isz, isub, 0)


def _sc_gather(table, gidx, d):
    nm, dpad = table.shape
    info = plsc.get_sparse_core_info()
    nw = info.num_cores * info.num_subcores
    b_per_w = nm // nw
    chunk = min(512, b_per_w)
    mesh = plsc.VectorSubcoreMesh(core_axis_name="c", subcore_axis_name="s")

    @functools.partial(
        pl.kernel, mesh=mesh,
        out_type=jax.ShapeDtypeStruct((nm, dpad), jnp.float32),
        scratch_types=[
            pltpu.VMEM((chunk,), jnp.int32),
            pltpu.VMEM((chunk, dpad), jnp.float32),
            pltpu.SemaphoreType.DMA,
        ],
    )
    def k(table_hbm, idx_hbm, out_hbm, idx_v, rows_v, sem):
        wid = lax.axis_index("s") * info.num_cores + lax.axis_index("c")
        base = wid * b_per_w
        for t in range(b_per_w // chunk):
            off = base + t * chunk
            pltpu.sync_copy(idx_hbm.at[pl.ds(off, chunk)], idx_v)
            pltpu.async_copy(table_hbm.at[idx_v], rows_v, sem).wait()
            pltpu.sync_copy(rows_v, out_hbm.at[pl.ds(off, chunk)])

    return k(table, gidx)[:, 0:d]


def kernel(states_prev, log_weights_prev, observations, controls, A, B, C):
    n, m, d = states_prev.shape
    od = observations.shape[1]
    cd = controls.shape[1]
    f32 = jnp.float32

    noise = jax.random.normal(jax.random.key(42), (n, m, d), dtype=f32) * 0.05
    keys = lax.bitcast_convert_type(
        jax.random.key_data(jax.random.split(jax.random.key(7), n)), jnp.int32)

    ck = min(2048, m)
    spred, cw, est = pl.pallas_call(
        functools.partial(_prep_body, m, d, ck),
        grid=(n,),
        in_specs=[
            pl.BlockSpec((1, m, d), lambda r: (r, 0, 0)),
            pl.BlockSpec((1, m, d), lambda r: (r, 0, 0)),
            pl.BlockSpec((1, 1, m), lambda r: (r, 0, 0)),
            pl.BlockSpec((1, 1, od), lambda r: (r, 0, 0)),
            pl.BlockSpec((1, 1, cd), lambda r: (r, 0, 0)),
            pl.BlockSpec((d, d), lambda r: (0, 0)),
            pl.BlockSpec((cd, d), lambda r: (0, 0)),
            pl.BlockSpec((d, od), lambda r: (0, 0)),
        ],
        out_specs=[
            pl.BlockSpec((1, m, 128), lambda r: (r, 0, 0)),
            pl.BlockSpec((1, 1, m), lambda r: (r, 0, 0)),
            pl.BlockSpec((1, 1, d), lambda r: (r, 0, 0)),
        ],
        out_shape=[
            jax.ShapeDtypeStruct((n, m, 128), f32),
            jax.ShapeDtypeStruct((n, 1, m), f32),
            jax.ShapeDtypeStruct((n, 1, d), f32),
        ],
        scratch_shapes=[pltpu.VMEM((1, m), f32)],
        compiler_params=pltpu.CompilerParams(
            dimension_semantics=("parallel",)),
    )(states_prev.reshape(n, m, d), noise,
      log_weights_prev.reshape(n, 1, m),
      observations.reshape(n, 1, od), controls.reshape(n, 1, cd), A, B, C)

    c3 = cw

    bi = min(512, m)
    isz = 16
    js = min(1024, m)
    gidx3 = pl.pallas_call(
        functools.partial(_sampler_body, m, bi, isz, js),
        grid=(n, m // bi),
        in_specs=[
            pl.BlockSpec(memory_space=pltpu.SMEM),
            pl.BlockSpec((1, 1, m), lambda r, ib: (r, 0, 0)),
        ],
        out_specs=pl.BlockSpec((1, bi, 1), lambda r, ib: (r, ib, 0)),
        out_shape=jax.ShapeDtypeStruct((n, m, 1), jnp.int32),
        compiler_params=pltpu.CompilerParams(
            dimension_semantics=("parallel", "parallel")),
    )(keys, c3)

    states = _sc_gather(spred.reshape(n * m, 128),
                        gidx3.reshape(n * m), d).reshape(n, m, d)
    log_weights = jnp.zeros((n, m), f32) - jnp.log(m)
    return (est.reshape(n, d), states, log_weights)


# speedup vs baseline: 1.6033x; 1.6033x over previous
import functools

import jax
import jax.numpy as jnp
import numpy as np
from jax import lax
from jax.experimental import pallas as pl
from jax.experimental.pallas import tpu as pltpu
from jax.experimental.pallas import tpu_sc as plsc

_INV2_23 = np.float32(2.0 ** -23)


def _tf_bits(k1, k2, p):
    ks2 = k1 ^ k2 ^ jnp.uint32(0x1BD11BDA)
    x0 = k1
    x1 = p + k2
    ks_a = (k2, ks2, k1, k2, ks2)
    ks_b = (ks2 + jnp.uint32(1), k1 + jnp.uint32(2), k2 + jnp.uint32(3),
            ks2 + jnp.uint32(4), k1 + jnp.uint32(5))
    rots = ((13, 15, 26, 6), (17, 29, 16, 24))
    for i in range(5):
        for r in rots[i & 1]:
            x0 = x0 + x1
            x1 = (x1 << r) | (x1 >> (32 - r))
            x1 = x0 ^ x1
        x0 = x0 + ks_a[i]
        x1 = x1 + ks_b[i]
    return x0 ^ x1


def _bf(x):
    return x.astype(jnp.bfloat16)


def _dot(a, b):
    return lax.dot_general(_bf(a), _bf(b), (((1,), (0,)), ((), ())),
                           preferred_element_type=jnp.float32)


def _prep_body(m, d, ck, sp_ref, noise_ref, lw_ref, obs_ref, ctrl_ref,
               a_ref, b_ref, c_ref, spred_ref, cw_ref, est_ref, lwp_ref):
    cb = _dot(ctrl_ref[0], b_ref[...])
    amat = a_ref[...]
    cmat = c_ref[...]
    obs = obs_ref[0]

    def pass1(t, _):
        sl = pl.ds(t * ck, ck)
        sp = sp_ref[0, sl, :]
        spred = sp + jnp.tanh(_dot(sp, amat) + cb) + noise_ref[0, sl, :]
        spred_ref[0, sl, 0:d] = spred
        po = _dot(spred, cmat)
        df = obs - po
        ll = -0.5 * jnp.sum(df * df, axis=1, keepdims=True)
        lwp_ref[:, sl] = lw_ref[0, :, sl] + jnp.reshape(ll, (1, ck))
        return 0

    lax.fori_loop(0, m // ck, pass1, 0)

    lwp = lwp_ref[...]
    mx = jnp.max(lwp)
    lse = mx + jnp.log(jnp.sum(jnp.exp(lwp - mx)))
    cw_ref[0] = jnp.exp(mx - lwp)

    def pass2(t, acc):
        sl = pl.ds(t * ck, ck)
        w = jnp.exp(lwp_ref[:, sl] - lse)
        return acc + lax.dot_general(
            w, spred_ref[0, sl, 0:d], (((1,), (0,)), ((), ())),
            preferred_element_type=jnp.float32,
            precision=lax.Precision.HIGHEST)

    est_ref[0] = lax.fori_loop(0, m // ck, pass2, jnp.zeros((1, d), jnp.float32))


def _sampler_body(m, bi, isz, js, keys_ref, c_ref, idx_ref):
    r = pl.program_id(0)
    ib = pl.program_id(1)
    k1 = keys_ref[r, 0].astype(jnp.uint32)
    k2 = keys_ref[r, 1].astype(jnp.uint32)
    i0 = ib * bi
    ii = lax.broadcasted_iota(jnp.int32, (isz, js), 0)
    jj = lax.broadcasted_iota(jnp.int32, (isz, js), 1)

    def isub(s, _):
        pbase = ((i0 + s * isz + ii) * m + jj).astype(jnp.uint32)

        def chunk(q, carry):
            rmin, ridx = carry
            p = pbase + (q * js).astype(jnp.uint32)
            bits = _tf_bits(k1, k2, p)
            mant = (bits >> 9).astype(jnp.int32)
            u = mant.astype(jnp.float32) * _INV2_23
            t = jnp.log2(u)
            cq = c_ref[0, :, pl.ds(q * js, js)]
            score = t * cq
            cm = jnp.max(score, axis=1, keepdims=True)
            jm = jnp.min(jnp.where(score == cm, jj + q * js, m),
                         axis=1, keepdims=True)
            upd = cm > rmin
            return jnp.where(upd, cm, rmin), jnp.where(upd, jm, ridx)

        rmin0 = jnp.full((isz, 1), -jnp.inf, jnp.float32)
        ridx0 = jnp.zeros((isz, 1), jnp.int32)
        _, ridx = lax.fori_loop(0, m // js, chunk, (rmin0, ridx0))
        idx_ref[0, pl.ds(s * isz, isz), :] = ridx + r * m
        return 0

    lax.fori_loop(0, bi // isz, isub, 0)


def _sc_gather(table, gidx, d):
    nm, dpad = table.shape
    info = plsc.get_sparse_core_info()
    nw = info.num_cores * info.num_subcores
    b_per_w = nm // nw
    chunk = min(512, b_per_w)
    mesh = plsc.VectorSubcoreMesh(core_axis_name="c", subcore_axis_name="s")

    @functools.partial(
        pl.kernel, mesh=mesh,
        out_type=jax.ShapeDtypeStruct((nm, dpad), jnp.float32),
        scratch_types=[
            pltpu.VMEM((chunk,), jnp.int32),
            pltpu.VMEM((chunk, dpad), jnp.float32),
            pltpu.SemaphoreType.DMA,
        ],
    )
    def k(table_hbm, idx_hbm, out_hbm, idx_v, rows_v, sem):
        wid = lax.axis_index("s") * info.num_cores + lax.axis_index("c")
        base = wid * b_per_w
        for t in range(b_per_w // chunk):
            off = base + t * chunk
            pltpu.sync_copy(idx_hbm.at[pl.ds(off, chunk)], idx_v)
            pltpu.async_copy(table_hbm.at[idx_v], rows_v, sem).wait()
            pltpu.sync_copy(rows_v, out_hbm.at[pl.ds(off, chunk)])

    return k(table, gidx)[:, 0:d]


def kernel(states_prev, log_weights_prev, observations, controls, A, B, C):
    n, m, d = states_prev.shape
    od = observations.shape[1]
    cd = controls.shape[1]
    f32 = jnp.float32

    noise = jax.random.normal(jax.random.key(42), (n, m, d), dtype=f32) * 0.05
    keys = lax.bitcast_convert_type(
        jax.random.key_data(jax.random.split(jax.random.key(7), n)), jnp.int32)

    ck = min(2048, m)
    spred, cw, est = pl.pallas_call(
        functools.partial(_prep_body, m, d, ck),
        grid=(n,),
        in_specs=[
            pl.BlockSpec((1, m, d), lambda r: (r, 0, 0)),
            pl.BlockSpec((1, m, d), lambda r: (r, 0, 0)),
            pl.BlockSpec((1, 1, m), lambda r: (r, 0, 0)),
            pl.BlockSpec((1, 1, od), lambda r: (r, 0, 0)),
            pl.BlockSpec((1, 1, cd), lambda r: (r, 0, 0)),
            pl.BlockSpec((d, d), lambda r: (0, 0)),
            pl.BlockSpec((cd, d), lambda r: (0, 0)),
            pl.BlockSpec((d, od), lambda r: (0, 0)),
        ],
        out_specs=[
            pl.BlockSpec((1, m, 128), lambda r: (r, 0, 0)),
            pl.BlockSpec((1, 1, m), lambda r: (r, 0, 0)),
            pl.BlockSpec((1, 1, d), lambda r: (r, 0, 0)),
        ],
        out_shape=[
            jax.ShapeDtypeStruct((n, m, 128), f32),
            jax.ShapeDtypeStruct((n, 1, m), f32),
            jax.ShapeDtypeStruct((n, 1, d), f32),
        ],
        scratch_shapes=[pltpu.VMEM((1, m), f32)],
        compiler_params=pltpu.CompilerParams(
            dimension_semantics=("parallel",)),
    )(states_prev.reshape(n, m, d), noise,
      log_weights_prev.reshape(n, 1, m),
      observations.reshape(n, 1, od), controls.reshape(n, 1, cd), A, B, C)

    c3 = cw

    bi = min(512, m)
    isz = 64
    js = min(1024, m)
    gidx3 = pl.pallas_call(
        functools.partial(_sampler_body, m, bi, isz, js),
        grid=(n, m // bi),
        in_specs=[
            pl.BlockSpec(memory_space=pltpu.SMEM),
            pl.BlockSpec((1, 1, m), lambda r, ib: (r, 0, 0)),
        ],
        out_specs=pl.BlockSpec((1, bi, 1), lambda r, ib: (r, ib, 0)),
        out_shape=jax.ShapeDtypeStruct((n, m, 1), jnp.int32),
        compiler_params=pltpu.CompilerParams(
            dimension_semantics=("parallel", "parallel")),
    )(keys, c3)

    states = _sc_gather(spred.reshape(n * m, 128),
                        gidx3.reshape(n * m), d).reshape(n, m, d)
    log_weights = jnp.zeros((n, m), f32) - jnp.log(m)
    return (est.reshape(n, d), states, log_weights)


# speedup vs baseline: 1.7897x; 1.1162x over previous
import functools

import jax
import jax.numpy as jnp
import numpy as np
from jax import lax
from jax.experimental import pallas as pl
from jax.experimental.pallas import tpu as pltpu
from jax.experimental.pallas import tpu_sc as plsc

_INV2_23 = np.float32(2.0 ** -23)


def _tf_bits(k1, k2, p):
    ks2 = k1 ^ k2 ^ jnp.uint32(0x1BD11BDA)
    x0 = k1
    x1 = p
    ks_a = (k2, ks2, k1, k2, ks2)
    ks_b = (ks2 + jnp.uint32(1), k1 + jnp.uint32(2), k2 + jnp.uint32(3),
            ks2 + jnp.uint32(4), k1 + jnp.uint32(5))
    rots = ((13, 15, 26, 6), (17, 29, 16, 24))
    for i in range(5):
        for r in rots[i & 1]:
            x0 = x0 + x1
            x1 = (x1 << r) | (x1 >> (32 - r))
            x1 = x0 ^ x1
        x0 = x0 + ks_a[i]
        x1 = x1 + ks_b[i]
    return x0 ^ x1


def _bf(x):
    return x.astype(jnp.bfloat16)


def _dot(a, b):
    return lax.dot_general(_bf(a), _bf(b), (((1,), (0,)), ((), ())),
                           preferred_element_type=jnp.float32)


def _prep_body(m, d, ck, sp_ref, noise_ref, lw_ref, obs_ref, ctrl_ref,
               a_ref, b_ref, c_ref, spred_ref, cw_ref, est_ref, lwp_ref):
    cb = _dot(ctrl_ref[0], b_ref[...])
    amat = a_ref[...]
    cmat = c_ref[...]
    obs = obs_ref[0]

    def pass1(t, _):
        sl = pl.ds(t * ck, ck)
        sp = sp_ref[0, sl, :]
        spred = sp + jnp.tanh(_dot(sp, amat) + cb) + noise_ref[0, sl, :]
        spred_ref[0, sl, 0:d] = spred
        po = _dot(spred, cmat)
        df = obs - po
        ll = -0.5 * jnp.sum(df * df, axis=1, keepdims=True)
        lwp_ref[:, sl] = lw_ref[0, :, sl] + jnp.reshape(ll, (1, ck))
        return 0

    lax.fori_loop(0, m // ck, pass1, 0)

    lwp = lwp_ref[...]
    mx = jnp.max(lwp)
    lse = mx + jnp.log(jnp.sum(jnp.exp(lwp - mx)))
    cw_ref[0] = jnp.exp(mx - lwp)

    def pass2(t, acc):
        sl = pl.ds(t * ck, ck)
        w = jnp.exp(lwp_ref[:, sl] - lse)
        return acc + lax.dot_general(
            w, spred_ref[0, sl, 0:d], (((1,), (0,)), ((), ())),
            preferred_element_type=jnp.float32,
            precision=lax.Precision.HIGHEST)

    est_ref[0] = lax.fori_loop(0, m // ck, pass2, jnp.zeros((1, d), jnp.float32))


def _sampler_body(m, bi, isz, js, keys_ref, c_ref, idx_ref):
    r = pl.program_id(0)
    ib = pl.program_id(1)
    k1 = keys_ref[r, 0].astype(jnp.uint32)
    k2 = keys_ref[r, 1].astype(jnp.uint32)
    i0 = ib * bi
    ii = lax.broadcasted_iota(jnp.int32, (isz, js), 0)
    jj = lax.broadcasted_iota(jnp.int32, (isz, js), 1)

    def isub(s, _):
        pbase = ((i0 + s * isz + ii) * m + jj).astype(jnp.uint32) + k2

        def chunk(q, carry):
            rmin, ridx = carry
            p = pbase + (q * js).astype(jnp.uint32)
            bits = _tf_bits(k1, k2, p)
            mant = (bits >> 9).astype(jnp.int32)
            u = mant.astype(jnp.float32) * _INV2_23
            t = jnp.log2(u)
            cq = c_ref[0, :, pl.ds(q * js, js)]
            score = t * cq
            cm = jnp.max(score, axis=1, keepdims=True)
            jm = jnp.min(jnp.where(score == cm, jj + q * js, m),
                         axis=1, keepdims=True)
            upd = cm > rmin
            return jnp.where(upd, cm, rmin), jnp.where(upd, jm, ridx)

        rmin0 = jnp.full((isz, 1), -jnp.inf, jnp.float32)
        ridx0 = jnp.zeros((isz, 1), jnp.int32)
        _, ridx = lax.fori_loop(0, m // js, chunk, (rmin0, ridx0),
                                unroll=2)
        idx_ref[0, pl.ds(s * isz, isz), :] = ridx + r * m
        return 0

    lax.fori_loop(0, bi // isz, isub, 0)


def _sc_gather(table, gidx, d):
    nm, dpad = table.shape
    info = plsc.get_sparse_core_info()
    nw = info.num_cores * info.num_subcores
    b_per_w = nm // nw
    chunk = min(512, b_per_w)
    mesh = plsc.VectorSubcoreMesh(core_axis_name="c", subcore_axis_name="s")

    @functools.partial(
        pl.kernel, mesh=mesh,
        out_type=jax.ShapeDtypeStruct((nm, dpad), jnp.float32),
        scratch_types=[
            pltpu.VMEM((chunk,), jnp.int32),
            pltpu.VMEM((chunk, dpad), jnp.float32),
            pltpu.SemaphoreType.DMA,
        ],
    )
    def k(table_hbm, idx_hbm, out_hbm, idx_v, rows_v, sem):
        wid = lax.axis_index("s") * info.num_cores + lax.axis_index("c")
        base = wid * b_per_w
        for t in range(b_per_w // chunk):
            off = base + t * chunk
            pltpu.sync_copy(idx_hbm.at[pl.ds(off, chunk)], idx_v)
            pltpu.async_copy(table_hbm.at[idx_v], rows_v, sem).wait()
            pltpu.sync_copy(rows_v, out_hbm.at[pl.ds(off, chunk)])

    return k(table, gidx)[:, 0:d]


def kernel(states_prev, log_weights_prev, observations, controls, A, B, C):
    n, m, d = states_prev.shape
    od = observations.shape[1]
    cd = controls.shape[1]
    f32 = jnp.float32

    noise = jax.random.normal(jax.random.key(42), (n, m, d), dtype=f32) * 0.05
    keys = lax.bitcast_convert_type(
        jax.random.key_data(jax.random.split(jax.random.key(7), n)), jnp.int32)

    ck = min(2048, m)
    spred, cw, est = pl.pallas_call(
        functools.partial(_prep_body, m, d, ck),
        grid=(n,),
        in_specs=[
            pl.BlockSpec((1, m, d), lambda r: (r, 0, 0)),
            pl.BlockSpec((1, m, d), lambda r: (r, 0, 0)),
            pl.BlockSpec((1, 1, m), lambda r: (r, 0, 0)),
            pl.BlockSpec((1, 1, od), lambda r: (r, 0, 0)),
            pl.BlockSpec((1, 1, cd), lambda r: (r, 0, 0)),
            pl.BlockSpec((d, d), lambda r: (0, 0)),
            pl.BlockSpec((cd, d), lambda r: (0, 0)),
            pl.BlockSpec((d, od), lambda r: (0, 0)),
        ],
        out_specs=[
            pl.BlockSpec((1, m, 128), lambda r: (r, 0, 0)),
            pl.BlockSpec((1, 1, m), lambda r: (r, 0, 0)),
            pl.BlockSpec((1, 1, d), lambda r: (r, 0, 0)),
        ],
        out_shape=[
            jax.ShapeDtypeStruct((n, m, 128), f32),
            jax.ShapeDtypeStruct((n, 1, m), f32),
            jax.ShapeDtypeStruct((n, 1, d), f32),
        ],
        scratch_shapes=[pltpu.VMEM((1, m), f32)],
        compiler_params=pltpu.CompilerParams(
            dimension_semantics=("parallel",)),
    )(states_prev.reshape(n, m, d), noise,
      log_weights_prev.reshape(n, 1, m),
      observations.reshape(n, 1, od), controls.reshape(n, 1, cd), A, B, C)

    c3 = cw

    bi = min(512, m)
    isz = 64
    js = min(1024, m)
    gidx3 = pl.pallas_call(
        functools.partial(_sampler_body, m, bi, isz, js),
        grid=(n, m // bi),
        in_specs=[
            pl.BlockSpec(memory_space=pltpu.SMEM),
            pl.BlockSpec((1, 1, m), lambda r, ib: (r, 0, 0)),
        ],
        out_specs=pl.BlockSpec((1, bi, 1), lambda r, ib: (r, ib, 0)),
        out_shape=jax.ShapeDtypeStruct((n, m, 1), jnp.int32),
        compiler_params=pltpu.CompilerParams(
            dimension_semantics=("parallel", "parallel")),
    )(keys, c3)

    states = _sc_gather(spred.reshape(n * m, 128),
                        gidx3.reshape(n * m), d).reshape(n, m, d)
    log_weights = jnp.zeros((n, m), f32) - jnp.log(m)
    return (est.reshape(n, d), states, log_weights)


# speedup vs baseline: 1.8727x; 1.0464x over previous
import functools

import jax
import jax.numpy as jnp
import numpy as np
from jax import lax
from jax.experimental import pallas as pl
from jax.experimental.pallas import tpu as pltpu
from jax.experimental.pallas import tpu_sc as plsc

_INV2_23 = np.float32(2.0 ** -23)


def _tf_bits(k1, k2, p):
    ks2 = k1 ^ k2 ^ jnp.uint32(0x1BD11BDA)
    x0 = k1
    x1 = p
    ks_a = (k2, ks2, k1, k2, ks2)
    ks_b = (ks2 + jnp.uint32(1), k1 + jnp.uint32(2), k2 + jnp.uint32(3),
            ks2 + jnp.uint32(4), k1 + jnp.uint32(5))
    rots = ((13, 15, 26, 6), (17, 29, 16, 24))
    for i in range(5):
        for r in rots[i & 1]:
            x0 = x0 + x1
            x1 = (x1 << r) | (x1 >> (32 - r))
            x1 = x0 ^ x1
        x0 = x0 + ks_a[i]
        x1 = x1 + ks_b[i]
    return x0 ^ x1


def _bf(x):
    return x.astype(jnp.bfloat16)


def _dot(a, b):
    return lax.dot_general(_bf(a), _bf(b), (((1,), (0,)), ((), ())),
                           preferred_element_type=jnp.float32)


def _prep_body(m, d, ck, sp_ref, noise_ref, lw_ref, obs_ref, ctrl_ref,
               a_ref, b_ref, c_ref, spred_ref, cw_ref, est_ref, lwp_ref):
    cb = _dot(ctrl_ref[0], b_ref[...])
    amat = a_ref[...]
    cmat = c_ref[...]
    obs = obs_ref[0]

    def pass1(t, _):
        sl = pl.ds(t * ck, ck)
        sp = sp_ref[0, sl, :]
        spred = sp + jnp.tanh(_dot(sp, amat) + cb) + noise_ref[0, sl, :]
        spred_ref[0, sl, 0:d] = spred
        po = _dot(spred, cmat)
        df = obs - po
        ll = -0.5 * jnp.sum(df * df, axis=1, keepdims=True)
        lwp_ref[:, sl] = lw_ref[0, :, sl] + jnp.reshape(ll, (1, ck))
        return 0

    lax.fori_loop(0, m // ck, pass1, 0)

    lwp = lwp_ref[...]
    mx = jnp.max(lwp)
    lse = mx + jnp.log(jnp.sum(jnp.exp(lwp - mx)))
    cw_ref[0] = jnp.exp(mx - lwp)

    def pass2(t, acc):
        sl = pl.ds(t * ck, ck)
        w = jnp.exp(lwp_ref[:, sl] - lse)
        return acc + lax.dot_general(
            w, spred_ref[0, sl, 0:d], (((1,), (0,)), ((), ())),
            preferred_element_type=jnp.float32,
            precision=lax.Precision.HIGHEST)

    est_ref[0] = lax.fori_loop(0, m // ck, pass2, jnp.zeros((1, d), jnp.float32))


def _sampler_body(m, bi, isz, js, keys_ref, c_ref, idx_ref):
    r = pl.program_id(0)
    ib = pl.program_id(1)
    k1 = keys_ref[r, 0].astype(jnp.uint32)
    k2 = keys_ref[r, 1].astype(jnp.uint32)
    i0 = ib * bi
    ii = lax.broadcasted_iota(jnp.int32, (isz, js), 0)
    jj = lax.broadcasted_iota(jnp.int32, (isz, js), 1)

    def isub(s, _):
        pbase = ((i0 + s * isz + ii) * m + jj).astype(jnp.uint32) + k2

        def chunk(q, carry):
            rmin, ridx = carry
            p = pbase + (q * js).astype(jnp.uint32)
            bits = _tf_bits(k1, k2, p)
            mant = (bits >> 9).astype(jnp.int32)
            u = mant.astype(jnp.float32) * _INV2_23
            t = jnp.log2(u)
            cq = c_ref[0, :, pl.ds(q * js, js)]
            score = t * cq
            cm = jnp.max(score, axis=1, keepdims=True)
            jm = jnp.min(jnp.where(score == cm, jj + q * js, m),
                         axis=1, keepdims=True)
            upd = cm > rmin
            return jnp.where(upd, cm, rmin), jnp.where(upd, jm, ridx)

        rmin0 = jnp.full((isz, 1), -jnp.inf, jnp.float32)
        ridx0 = jnp.zeros((isz, 1), jnp.int32)
        _, ridx = lax.fori_loop(0, m // js, chunk, (rmin0, ridx0),
                                unroll=4)
        idx_ref[0, pl.ds(s * isz, isz), :] = ridx + r * m
        return 0

    lax.fori_loop(0, bi // isz, isub, 0)


def _sc_gather(table, gidx, d):
    nm, dpad = table.shape
    info = plsc.get_sparse_core_info()
    nw = info.num_cores * info.num_subcores
    b_per_w = nm // nw
    chunk = min(512, b_per_w)
    mesh = plsc.VectorSubcoreMesh(core_axis_name="c", subcore_axis_name="s")

    @functools.partial(
        pl.kernel, mesh=mesh,
        out_type=jax.ShapeDtypeStruct((nm, dpad), jnp.float32),
        scratch_types=[
            pltpu.VMEM((chunk,), jnp.int32),
            pltpu.VMEM((chunk, dpad), jnp.float32),
            pltpu.SemaphoreType.DMA,
        ],
    )
    def k(table_hbm, idx_hbm, out_hbm, idx_v, rows_v, sem):
        wid = lax.axis_index("s") * info.num_cores + lax.axis_index("c")
        base = wid * b_per_w
        for t in range(b_per_w // chunk):
            off = base + t * chunk
            pltpu.sync_copy(idx_hbm.at[pl.ds(off, chunk)], idx_v)
            pltpu.async_copy(table_hbm.at[idx_v], rows_v, sem).wait()
            pltpu.sync_copy(rows_v, out_hbm.at[pl.ds(off, chunk)])

    return k(table, gidx)[:, 0:d]


def kernel(states_prev, log_weights_prev, observations, controls, A, B, C):
    n, m, d = states_prev.shape
    od = observations.shape[1]
    cd = controls.shape[1]
    f32 = jnp.float32

    noise = jax.random.normal(jax.random.key(42), (n, m, d), dtype=f32) * 0.05
    keys = lax.bitcast_convert_type(
        jax.random.key_data(jax.random.split(jax.random.key(7), n)), jnp.int32)

    ck = min(2048, m)
    spred, cw, est = pl.pallas_call(
        functools.partial(_prep_body, m, d, ck),
        grid=(n,),
        in_specs=[
            pl.BlockSpec((1, m, d), lambda r: (r, 0, 0)),
            pl.BlockSpec((1, m, d), lambda r: (r, 0, 0)),
            pl.BlockSpec((1, 1, m), lambda r: (r, 0, 0)),
            pl.BlockSpec((1, 1, od), lambda r: (r, 0, 0)),
            pl.BlockSpec((1, 1, cd), lambda r: (r, 0, 0)),
            pl.BlockSpec((d, d), lambda r: (0, 0)),
            pl.BlockSpec((cd, d), lambda r: (0, 0)),
            pl.BlockSpec((d, od), lambda r: (0, 0)),
        ],
        out_specs=[
            pl.BlockSpec((1, m, 128), lambda r: (r, 0, 0)),
            pl.BlockSpec((1, 1, m), lambda r: (r, 0, 0)),
            pl.BlockSpec((1, 1, d), lambda r: (r, 0, 0)),
        ],
        out_shape=[
            jax.ShapeDtypeStruct((n, m, 128), f32),
            jax.ShapeDtypeStruct((n, 1, m), f32),
            jax.ShapeDtypeStruct((n, 1, d), f32),
        ],
        scratch_shapes=[pltpu.VMEM((1, m), f32)],
        compiler_params=pltpu.CompilerParams(
            dimension_semantics=("parallel",)),
    )(states_prev.reshape(n, m, d), noise,
      log_weights_prev.reshape(n, 1, m),
      observations.reshape(n, 1, od), controls.reshape(n, 1, cd), A, B, C)

    c3 = cw

    bi = min(2048, m)
    isz = 64
    js = min(1024, m)
    gidx3 = pl.pallas_call(
        functools.partial(_sampler_body, m, bi, isz, js),
        grid=(n, m // bi),
        in_specs=[
            pl.BlockSpec(memory_space=pltpu.SMEM),
            pl.BlockSpec((1, 1, m), lambda r, ib: (r, 0, 0)),
        ],
        out_specs=pl.BlockSpec((1, bi, 1), lambda r, ib: (r, ib, 0)),
        out_shape=jax.ShapeDtypeStruct((n, m, 1), jnp.int32),
        compiler_params=pltpu.CompilerParams(
            dimension_semantics=("parallel", "parallel")),
    )(keys, c3)

    states = _sc_gather(spred.reshape(n * m, 128),
                        gidx3.reshape(n * m), d).reshape(n, m, d)
    log_weights = jnp.zeros((n, m), f32) - jnp.log(m)
    return (est.reshape(n, d), states, log_weights)


# speedup vs baseline: 1.9065x; 1.0180x over previous
import functools

import jax
import jax.numpy as jnp
import numpy as np
from jax import lax
from jax.experimental import pallas as pl
from jax.experimental.pallas import tpu as pltpu
from jax.experimental.pallas import tpu_sc as plsc

_INV2_23 = np.float32(2.0 ** -23)


def _tf_bits(k1, k2, p):
    ks2 = k1 ^ k2 ^ jnp.uint32(0x1BD11BDA)
    x0 = k1
    x1 = p
    ks_a = (k2, ks2, k1, k2, ks2)
    ks_b = (ks2 + jnp.uint32(1), k1 + jnp.uint32(2), k2 + jnp.uint32(3),
            ks2 + jnp.uint32(4), k1 + jnp.uint32(5))
    rots = ((13, 15, 26, 6), (17, 29, 16, 24))
    for i in range(5):
        for r in rots[i & 1]:
            x0 = x0 + x1
            x1 = (x1 << r) | (x1 >> (32 - r))
            x1 = x0 ^ x1
        x0 = x0 + ks_a[i]
        x1 = x1 + ks_b[i]
    return x0 ^ x1


def _bf(x):
    return x.astype(jnp.bfloat16)


def _dot(a, b):
    return lax.dot_general(_bf(a), _bf(b), (((1,), (0,)), ((), ())),
                           preferred_element_type=jnp.float32)


def _prep_body(m, d, ck, sp_ref, noise_ref, lw_ref, obs_ref, ctrl_ref,
               a_ref, b_ref, c_ref, spred_ref, cw_ref, est_ref, lwp_ref):
    cb = _dot(ctrl_ref[0], b_ref[...])
    amat = a_ref[...]
    cmat = c_ref[...]
    obs = obs_ref[0]

    def pass1(t, _):
        sl = pl.ds(t * ck, ck)
        sp = sp_ref[0, sl, :]
        spred = sp + jnp.tanh(_dot(sp, amat) + cb) + noise_ref[0, sl, :]
        spred_ref[0, sl, 0:d] = spred
        po = _dot(spred, cmat)
        df = obs - po
        ll = -0.5 * jnp.sum(df * df, axis=1, keepdims=True)
        lwp_ref[:, sl] = lw_ref[0, :, sl] + jnp.reshape(ll, (1, ck))
        return 0

    lax.fori_loop(0, m // ck, pass1, 0)

    lwp = lwp_ref[...]
    mx = jnp.max(lwp)
    lse = mx + jnp.log(jnp.sum(jnp.exp(lwp - mx)))
    cw_ref[0] = jnp.exp(mx - lwp)

    def pass2(t, acc):
        sl = pl.ds(t * ck, ck)
        w = jnp.exp(lwp_ref[:, sl] - lse)
        return acc + lax.dot_general(
            w, spred_ref[0, sl, 0:d], (((1,), (0,)), ((), ())),
            preferred_element_type=jnp.float32,
            precision=lax.Precision.HIGHEST)

    est_ref[0] = lax.fori_loop(0, m // ck, pass2, jnp.zeros((1, d), jnp.float32))


def _sampler_body(m, bi, isz, js, keys_ref, c_ref, idx_ref):
    r = pl.program_id(0)
    ib = pl.program_id(1)
    k1 = keys_ref[r, 0].astype(jnp.uint32)
    k2 = keys_ref[r, 1].astype(jnp.uint32)
    i0 = ib * bi
    ii = lax.broadcasted_iota(jnp.int32, (isz, js), 0)
    jj = lax.broadcasted_iota(jnp.int32, (isz, js), 1)

    def isub(s, _):
        pbase = ((i0 + s * isz + ii) * m + jj).astype(jnp.uint32) + k2

        def chunk(q, carry):
            rmin, ridx = carry
            p = pbase + (q * js).astype(jnp.uint32)
            bits = _tf_bits(k1, k2, p)
            mant = (bits >> 9).astype(jnp.int32)
            u = mant.astype(jnp.float32) * _INV2_23
            t = jnp.log2(u)
            cq = c_ref[0, :, pl.ds(q * js, js)]
            score = t * cq
            cm = jnp.max(score, axis=1, keepdims=True)
            jm = jnp.min(jnp.where(score == cm, jj + q * js, m),
                         axis=1, keepdims=True)
            upd = cm > rmin
            return jnp.where(upd, cm, rmin), jnp.where(upd, jm, ridx)

        rmin0 = jnp.full((isz, 1), -jnp.inf, jnp.float32)
        ridx0 = jnp.zeros((isz, 1), jnp.int32)
        _, ridx = lax.fori_loop(0, m // js, chunk, (rmin0, ridx0),
                                unroll=8)
        idx_ref[0, pl.ds(s * isz, isz), :] = ridx + r * m
        return 0

    lax.fori_loop(0, bi // isz, isub, 0)


def _sc_gather(table, gidx, d):
    nm, dpad = table.shape
    info = plsc.get_sparse_core_info()
    nw = info.num_cores * info.num_subcores
    b_per_w = nm // nw
    chunk = min(512, b_per_w)
    mesh = plsc.VectorSubcoreMesh(core_axis_name="c", subcore_axis_name="s")

    @functools.partial(
        pl.kernel, mesh=mesh,
        out_type=jax.ShapeDtypeStruct((nm, dpad), jnp.float32),
        scratch_types=[
            pltpu.VMEM((chunk,), jnp.int32),
            pltpu.VMEM((chunk, dpad), jnp.float32),
            pltpu.SemaphoreType.DMA,
        ],
    )
    def k(table_hbm, idx_hbm, out_hbm, idx_v, rows_v, sem):
        wid = lax.axis_index("s") * info.num_cores + lax.axis_index("c")
        base = wid * b_per_w
        for t in range(b_per_w // chunk):
            off = base + t * chunk
            pltpu.sync_copy(idx_hbm.at[pl.ds(off, chunk)], idx_v)
            pltpu.async_copy(table_hbm.at[idx_v], rows_v, sem).wait()
            pltpu.sync_copy(rows_v, out_hbm.at[pl.ds(off, chunk)])

    return k(table, gidx)[:, 0:d]


def kernel(states_prev, log_weights_prev, observations, controls, A, B, C):
    n, m, d = states_prev.shape
    od = observations.shape[1]
    cd = controls.shape[1]
    f32 = jnp.float32

    noise = jax.random.normal(jax.random.key(42), (n, m, d), dtype=f32) * 0.05
    keys = lax.bitcast_convert_type(
        jax.random.key_data(jax.random.split(jax.random.key(7), n)), jnp.int32)

    ck = min(2048, m)
    spred, cw, est = pl.pallas_call(
        functools.partial(_prep_body, m, d, ck),
        grid=(n,),
        in_specs=[
            pl.BlockSpec((1, m, d), lambda r: (r, 0, 0)),
            pl.BlockSpec((1, m, d), lambda r: (r, 0, 0)),
            pl.BlockSpec((1, 1, m), lambda r: (r, 0, 0)),
            pl.BlockSpec((1, 1, od), lambda r: (r, 0, 0)),
            pl.BlockSpec((1, 1, cd), lambda r: (r, 0, 0)),
            pl.BlockSpec((d, d), lambda r: (0, 0)),
            pl.BlockSpec((cd, d), lambda r: (0, 0)),
            pl.BlockSpec((d, od), lambda r: (0, 0)),
        ],
        out_specs=[
            pl.BlockSpec((1, m, 128), lambda r: (r, 0, 0)),
            pl.BlockSpec((1, 1, m), lambda r: (r, 0, 0)),
            pl.BlockSpec((1, 1, d), lambda r: (r, 0, 0)),
        ],
        out_shape=[
            jax.ShapeDtypeStruct((n, m, 128), f32),
            jax.ShapeDtypeStruct((n, 1, m), f32),
            jax.ShapeDtypeStruct((n, 1, d), f32),
        ],
        scratch_shapes=[pltpu.VMEM((1, m), f32)],
        compiler_params=pltpu.CompilerParams(
            dimension_semantics=("parallel",)),
    )(states_prev.reshape(n, m, d), noise,
      log_weights_prev.reshape(n, 1, m),
      observations.reshape(n, 1, od), controls.reshape(n, 1, cd), A, B, C)

    c3 = cw

    bi = m
    isz = 64
    js = min(1024, m)
    gidx3 = pl.pallas_call(
        functools.partial(_sampler_body, m, bi, isz, js),
        grid=(n, m // bi),
        in_specs=[
            pl.BlockSpec(memory_space=pltpu.SMEM),
            pl.BlockSpec((1, 1, m), lambda r, ib: (r, 0, 0)),
        ],
        out_specs=pl.BlockSpec((1, bi, 1), lambda r, ib: (r, ib, 0)),
        out_shape=jax.ShapeDtypeStruct((n, m, 1), jnp.int32),
        compiler_params=pltpu.CompilerParams(
            dimension_semantics=("parallel", "parallel")),
    )(keys, c3)

    states = _sc_gather(spred.reshape(n * m, 128),
                        gidx3.reshape(n * m), d).reshape(n, m, d)
    log_weights = jnp.zeros((n, m), f32) - jnp.log(m)
    return (est.reshape(n, d), states, log_weights)


# speedup vs baseline: 1.9113x; 1.0025x over previous
import functools

import jax
import jax.numpy as jnp
import numpy as np
from jax import lax
from jax.experimental import pallas as pl
from jax.experimental.pallas import tpu as pltpu
from jax.experimental.pallas import tpu_sc as plsc

_INV2_23 = np.float32(2.0 ** -23)


def _tf_bits(k1, k2, p):
    ks2 = k1 ^ k2 ^ jnp.uint32(0x1BD11BDA)
    x0 = k1
    x1 = p
    ks_a = (k2, ks2, k1, k2, ks2)
    ks_b = (ks2 + jnp.uint32(1), k1 + jnp.uint32(2), k2 + jnp.uint32(3),
            ks2 + jnp.uint32(4), k1 + jnp.uint32(5))
    rots = ((13, 15, 26, 6), (17, 29, 16, 24))
    for i in range(5):
        for r in rots[i & 1]:
            x0 = x0 + x1
            x1 = (x1 << r) | (x1 >> (32 - r))
            x1 = x0 ^ x1
        x0 = x0 + ks_a[i]
        x1 = x1 + ks_b[i]
    return x0 ^ x1


def _bf(x):
    return x.astype(jnp.bfloat16)


def _dot(a, b):
    return lax.dot_general(_bf(a), _bf(b), (((1,), (0,)), ((), ())),
                           preferred_element_type=jnp.float32)


def _prep_body(m, d, ck, sp_ref, noise_ref, lw_ref, obs_ref, ctrl_ref,
               a_ref, b_ref, c_ref, spred_ref, cw_ref, est_ref, lwp_ref):
    cb = _dot(ctrl_ref[0], b_ref[...])
    amat = a_ref[...]
    cmat = c_ref[...]
    obs = obs_ref[0]

    obs_t = jnp.reshape(obs, (obs.shape[1], 1))

    def pass1(t, _):
        sl = pl.ds(t * ck, ck)
        sp = sp_ref[0, sl, :]
        spred = sp + jnp.tanh(_dot(sp, amat) + cb) + noise_ref[0, sl, :]
        spred_ref[0, sl, 0:d] = spred
        po_t = lax.dot_general(
            _bf(cmat), _bf(spred), (((0,), (1,)), ((), ())),
            preferred_element_type=jnp.float32)
        df = obs_t - po_t
        ll = -0.5 * jnp.sum(df * df, axis=0, keepdims=True)
        lwp_ref[:, sl] = lw_ref[0, :, sl] + ll
        return 0

    lax.fori_loop(0, m // ck, pass1, 0)

    lwp = lwp_ref[...]
    mx = jnp.max(lwp)
    lse = mx + jnp.log(jnp.sum(jnp.exp(lwp - mx)))
    cw_ref[0] = jnp.exp(mx - lwp)

    def pass2(t, acc):
        sl = pl.ds(t * ck, ck)
        w = jnp.exp(lwp_ref[:, sl] - lse)
        return acc + lax.dot_general(
            w, spred_ref[0, sl, 0:d], (((1,), (0,)), ((), ())),
            preferred_element_type=jnp.float32,
            precision=lax.Precision.HIGHEST)

    est_ref[0] = lax.fori_loop(0, m // ck, pass2, jnp.zeros((1, d), jnp.float32))


def _sampler_body(m, bi, isz, js, keys_ref, c_ref, idx_ref):
    r = pl.program_id(0)
    ib = pl.program_id(1)
    k1 = keys_ref[r, 0].astype(jnp.uint32)
    k2 = keys_ref[r, 1].astype(jnp.uint32)
    i0 = ib * bi
    ii = lax.broadcasted_iota(jnp.int32, (isz, js), 0)
    jj = lax.broadcasted_iota(jnp.int32, (isz, js), 1)

    def isub(s, _):
        pbase = ((i0 + s * isz + ii) * m + jj).astype(jnp.uint32) + k2

        def chunk(q, carry):
            rmin, ridx = carry
            p = pbase + (q * js).astype(jnp.uint32)
            bits = _tf_bits(k1, k2, p)
            mant = (bits >> 9).astype(jnp.int32)
            u = mant.astype(jnp.float32) * _INV2_23
            t = jnp.log2(u)
            cq = c_ref[0, :, pl.ds(q * js, js)]
            score = t * cq
            cm = jnp.max(score, axis=1, keepdims=True)
            jm = jnp.min(jnp.where(score == cm, jj + q * js, m),
                         axis=1, keepdims=True)
            upd = cm > rmin
            return jnp.where(upd, cm, rmin), jnp.where(upd, jm, ridx)

        rmin0 = jnp.full((isz, 1), -jnp.inf, jnp.float32)
        ridx0 = jnp.zeros((isz, 1), jnp.int32)
        _, ridx = lax.fori_loop(0, m // js, chunk, (rmin0, ridx0),
                                unroll=8)
        idx_ref[0, pl.ds(s * isz, isz), :] = ridx + r * m
        return 0

    lax.fori_loop(0, bi // isz, isub, 0)


def _sc_gather(table, gidx, d):
    nm, dpad = table.shape
    info = plsc.get_sparse_core_info()
    nw = info.num_cores * info.num_subcores
    b_per_w = nm // nw
    chunk = min(512, b_per_w)
    mesh = plsc.VectorSubcoreMesh(core_axis_name="c", subcore_axis_name="s")

    @functools.partial(
        pl.kernel, mesh=mesh,
        out_type=jax.ShapeDtypeStruct((nm, dpad), jnp.float32),
        scratch_types=[
            pltpu.VMEM((chunk,), jnp.int32),
            pltpu.VMEM((chunk, dpad), jnp.float32),
            pltpu.SemaphoreType.DMA,
        ],
    )
    def k(table_hbm, idx_hbm, out_hbm, idx_v, rows_v, sem):
        wid = lax.axis_index("s") * info.num_cores + lax.axis_index("c")
        base = wid * b_per_w
        for t in range(b_per_w // chunk):
            off = base + t * chunk
            pltpu.sync_copy(idx_hbm.at[pl.ds(off, chunk)], idx_v)
            pltpu.async_copy(table_hbm.at[idx_v], rows_v, sem).wait()
            pltpu.sync_copy(rows_v, out_hbm.at[pl.ds(off, chunk)])

    return k(table, gidx)[:, 0:d]


def kernel(states_prev, log_weights_prev, observations, controls, A, B, C):
    n, m, d = states_prev.shape
    od = observations.shape[1]
    cd = controls.shape[1]
    f32 = jnp.float32

    noise = jax.random.normal(jax.random.key(42), (n, m, d), dtype=f32) * 0.05
    keys = lax.bitcast_convert_type(
        jax.random.key_data(jax.random.split(jax.random.key(7), n)), jnp.int32)

    ck = min(2048, m)
    spred, cw, est = pl.pallas_call(
        functools.partial(_prep_body, m, d, ck),
        grid=(n,),
        in_specs=[
            pl.BlockSpec((1, m, d), lambda r: (r, 0, 0)),
            pl.BlockSpec((1, m, d), lambda r: (r, 0, 0)),
            pl.BlockSpec((1, 1, m), lambda r: (r, 0, 0)),
            pl.BlockSpec((1, 1, od), lambda r: (r, 0, 0)),
            pl.BlockSpec((1, 1, cd), lambda r: (r, 0, 0)),
            pl.BlockSpec((d, d), lambda r: (0, 0)),
            pl.BlockSpec((cd, d), lambda r: (0, 0)),
            pl.BlockSpec((d, od), lambda r: (0, 0)),
        ],
        out_specs=[
            pl.BlockSpec((1, m, 128), lambda r: (r, 0, 0)),
            pl.BlockSpec((1, 1, m), lambda r: (r, 0, 0)),
            pl.BlockSpec((1, 1, d), lambda r: (r, 0, 0)),
        ],
        out_shape=[
            jax.ShapeDtypeStruct((n, m, 128), f32),
            jax.ShapeDtypeStruct((n, 1, m), f32),
            jax.ShapeDtypeStruct((n, 1, d), f32),
        ],
        scratch_shapes=[pltpu.VMEM((1, m), f32)],
        compiler_params=pltpu.CompilerParams(
            dimension_semantics=("parallel",)),
    )(states_prev.reshape(n, m, d), noise,
      log_weights_prev.reshape(n, 1, m),
      observations.reshape(n, 1, od), controls.reshape(n, 1, cd), A, B, C)

    c3 = cw

    bi = m
    isz = 64
    js = min(1024, m)
    gidx3 = pl.pallas_call(
        functools.partial(_sampler_body, m, bi, isz, js),
        grid=(n, m // bi),
        in_specs=[
            pl.BlockSpec(memory_space=pltpu.SMEM),
            pl.BlockSpec((1, 1, m), lambda r, ib: (r, 0, 0)),
        ],
        out_specs=pl.BlockSpec((1, bi, 1), lambda r, ib: (r, ib, 0)),
        out_shape=jax.ShapeDtypeStruct((n, m, 1), jnp.int32),
        compiler_params=pltpu.CompilerParams(
            dimension_semantics=("parallel", "parallel")),
    )(keys, c3)

    states = _sc_gather(spred.reshape(n * m, 128),
                        gidx3.reshape(n * m), d).reshape(n, m, d)
    log_weights = jnp.zeros((n, m), f32) - jnp.log(m)
    return (est.reshape(n, d), states, log_weights)
